# baseline (speedup 1.0000x reference)
import functools
import math

import jax
import jax.numpy as jnp
from jax import lax
from jax.experimental import pallas as pl
from jax.experimental.pallas import tpu as pltpu
from jax.experimental.pallas import tpu_sc as plsc

_CFGS = [[1, 16, 1, 1], [6, 24, 2, 2], [6, 32, 3, 2], [6, 64, 4, 2],
         [6, 96, 3, 1], [6, 160, 3, 2], [6, 320, 1, 1]]
_NUM_CLASSES = 21
_TOPK = 200
_IOU_THR = 0.5

_B = 8
_NPAD = 2816
_PER_W = 704
_CHUNKS = 44
_OUTPAD = 208
_WPI = 4
_BIG = 2 ** 30


def _make_specs():
    specs = [('conv', 3, 32, 3, 2, 1, 1)]
    in_ch = 32
    for t, c, n, s in _CFGS:
        for i in range(n):
            stride = s if i == 0 else 1
            specs.append(('ir', in_ch, c, stride, t))
            in_ch = c
    specs.append(('conv', in_ch, 1280, 1, 1, 0, 1))
    return specs


def _conv2d(x, w, stride, padding, groups=1):
    return jax.lax.conv_general_dilated(
        x, w, (stride, stride), [(padding, padding), (padding, padding)],
        dimension_numbers=('NCHW', 'OIHW', 'NCHW'), feature_group_count=groups)


def _bn(x, g, b, eps=1e-5):
    mean = jnp.mean(x, axis=(0, 2, 3), keepdims=True)
    var = jnp.var(x, axis=(0, 2, 3), keepdims=True)
    xn = (x - mean) / jnp.sqrt(var + eps)
    return xn * g[None, :, None, None] + b[None, :, None, None]


def _relu6(x):
    return jnp.clip(x, 0.0, 6.0)


def _ssd_forward(x, params):
    specs = _make_specs()
    feats = []
    for li, spec in enumerate(specs):
        if spec[0] == 'conv':
            _, ci, co, ks, st, pd, gr = spec
            x = _relu6(_bn(_conv2d(x, params[f'l{li}_w'], st, pd, gr),
                           params[f'l{li}_g'], params[f'l{li}_b']))
        else:
            _, ci, co, st, t = spec
            hid = ci * t
            h = x
            if t != 1:
                h = _relu6(_bn(_conv2d(h, params[f'l{li}_pw1_w'], 1, 0),
                               params[f'l{li}_pw1_g'], params[f'l{li}_pw1_b']))
            h = _relu6(_bn(_conv2d(h, params[f'l{li}_dw_w'], st, 1, groups=hid),
                           params[f'l{li}_dw_g'], params[f'l{li}_dw_b']))
            h = _bn(_conv2d(h, params[f'l{li}_pw2_w'], 1, 0),
                    params[f'l{li}_pw2_g'], params[f'l{li}_pw2_b'])
            if st == 1 and ci == co:
                h = x + h
            x = h
        if li == 13 or li == 17:
            feats.append(x)
    loc_list, cls_list = [], []
    for i, f in enumerate(feats):
        lp = _conv2d(f, params[f'loc{i}_w'], 1, 1) + params[f'loc{i}_b2'][None, :, None, None]
        cp = _conv2d(f, params[f'cls{i}_w'], 1, 1) + params[f'cls{i}_b2'][None, :, None, None]
        loc_list.append(jnp.transpose(lp, (0, 2, 3, 1)).reshape(lp.shape[0], -1))
        cls_list.append(jnp.transpose(cp, (0, 2, 3, 1)).reshape(cp.shape[0], -1))
    B = x.shape[0]
    loc = jnp.concatenate(loc_list, axis=1).reshape(B, -1, 4)
    cls = jnp.concatenate(cls_list, axis=1).reshape(B, -1, _NUM_CLASSES)
    return loc, cls


def _score_kernel(cls_ref, scores_ref, labels_ref):
    C = cls_ref.shape[0]
    scores = cls_ref[0]
    labels = jnp.zeros(scores.shape, jnp.int32)
    for k in range(1, C):
        cur = cls_ref[k]
        better = cur > scores
        scores = jnp.where(better, cur, scores)
        labels = jnp.where(better, k, labels)
    scores_ref[...] = scores
    labels_ref[...] = labels


def _nms_sc(boxes_hbm, scores_hbm, labels_hbm,
            ox1_hbm, oy1_hbm, ox2_hbm, oy2_hbm, olab_hbm, osc_hbm,
            x1v, y1v, x2v, y2v, labv, sv,
            stgv, stgiv, mrgv, mrgiv,
            o1v, o2v, o3v, o4v, olv, osv,
            stage_m, stage_g):
    neg_inf = jnp.float32(-jnp.inf)
    c = lax.axis_index("c")
    s = lax.axis_index("s")
    b = c * (_B // 2) + s // _WPI
    w = s % _WPI
    base = w * _PER_W
    lanes = lax.iota(jnp.int32, 16)
    lane0 = lanes == 0
    s0 = (s // _WPI) * _WPI

    boff = pl.multiple_of(b * _NPAD, 8)
    pltpu.sync_copy(boxes_hbm.at[pl.ds(pl.multiple_of((0 * _B + b) * _NPAD, 8), _NPAD)], x1v)
    pltpu.sync_copy(boxes_hbm.at[pl.ds(pl.multiple_of((1 * _B + b) * _NPAD, 8), _NPAD)], y1v)
    pltpu.sync_copy(boxes_hbm.at[pl.ds(pl.multiple_of((2 * _B + b) * _NPAD, 8), _NPAD)], x2v)
    pltpu.sync_copy(boxes_hbm.at[pl.ds(pl.multiple_of((3 * _B + b) * _NPAD, 8), _NPAD)], y2v)
    pltpu.sync_copy(labels_hbm.at[pl.ds(boff, _NPAD)], labv)
    pltpu.sync_copy(scores_hbm.at[pl.ds(pl.multiple_of(b * _NPAD + base, 8), _PER_W)], sv)

    def round_body(t, carry):
        def amax(cix, car):
            bm, bi = car
            v = sv[pl.ds(cix * 16, 16)]
            idx = base + cix * 16 + lanes
            upd = v > bm
            return (jnp.where(upd, v, bm), jnp.where(upd, idx, bi))
        bm, bi = lax.fori_loop(
            0, _CHUNKS, amax,
            (jnp.full((16,), neg_inf, jnp.float32),
             jnp.full((16,), _BIG, jnp.int32)), unroll=4)
        for sh in (8, 4, 2, 1):
            part = lanes ^ sh
            m2 = bm.at[part].get(mode='promise_in_bounds', unique_indices=True)
            g2 = bi.at[part].get(mode='promise_in_bounds', unique_indices=True)
            take = (m2 > bm) | ((m2 == bm) & (g2 < bi))
            bm = jnp.where(take, m2, bm)
            bi = jnp.where(take, g2, bi)

        p = t % 2
        stgv[...] = bm
        stgiv[...] = bi
        woff = pl.multiple_of(p * 256 + s * 16, 8)
        roff = pl.multiple_of(p * 256 + s0 * 16, 8)
        pltpu.sync_copy(stgv, stage_m.at[pl.ds(woff, 16)])
        pltpu.sync_copy(stgiv, stage_g.at[pl.ds(woff, 16)])
        plsc.subcore_barrier()
        pltpu.sync_copy(stage_m.at[pl.ds(roff, 64)], mrgv)
        pltpu.sync_copy(stage_g.at[pl.ds(roff, 64)], mrgiv)

        bmv = mrgv[pl.ds(0, 16)]
        bgv = mrgiv[pl.ds(0, 16)]
        for k in range(1, _WPI):
            mk = mrgv[pl.ds(k * 16, 16)]
            gk = mrgiv[pl.ds(k * 16, 16)]
            take = (mk > bmv) | ((mk == bmv) & (gk < bgv))
            bmv = jnp.where(take, mk, bmv)
            bgv = jnp.where(take, gk, bgv)
        anyv = bmv != neg_inf
        giv = jnp.where(anyv, bgv, 0)

        cx1 = plsc.load_gather(x1v, [giv])
        cy1 = plsc.load_gather(y1v, [giv])
        cx2 = plsc.load_gather(x2v, [giv])
        cy2 = plsc.load_gather(y2v, [giv])
        car = (cx2 - cx1) * (cy2 - cy1)

        def sup(cix, _):
            off = base + cix * 16
            vx1 = x1v[pl.ds(off, 16)]
            vy1 = y1v[pl.ds(off, 16)]
            vx2 = x2v[pl.ds(off, 16)]
            vy2 = y2v[pl.ds(off, 16)]
            ss = sv[pl.ds(cix * 16, 16)]
            xx1 = jnp.maximum(cx1, vx1)
            yy1 = jnp.maximum(cy1, vy1)
            xx2 = jnp.minimum(cx2, vx2)
            yy2 = jnp.minimum(cy2, vy2)
            wd = jnp.maximum(xx2 - xx1, 0.0)
            ht = jnp.maximum(yy2 - yy1, 0.0)
            inter = wd * ht
            ar = (vx2 - vx1) * (vy2 - vy1)
            union = car + ar - inter
            iou = inter / union
            idx = off + lanes
            keep = (iou <= _IOU_THR) & (idx != giv)
            sv[pl.ds(cix * 16, 16)] = jnp.where(keep, ss, neg_inf)
            return 0
        lax.fori_loop(0, _CHUNKS, sup, 0, unroll=2)

        @pl.when(w == 0)
        def _():
            mm = jnp.where(anyv, jnp.float32(1.0), jnp.float32(0.0))
            mi = jnp.where(anyv, 1, 0)
            clab = plsc.load_gather(labv, [giv])
            tv = jnp.full((16,), t, jnp.int32)
            plsc.store_scatter(o1v, [tv], cx1 * mm, mask=lane0)
            plsc.store_scatter(o2v, [tv], cy1 * mm, mask=lane0)
            plsc.store_scatter(o3v, [tv], cx2 * mm, mask=lane0)
            plsc.store_scatter(o4v, [tv], cy2 * mm, mask=lane0)
            plsc.store_scatter(olv, [tv], clab * mi, mask=lane0)
            plsc.store_scatter(osv, [tv], jnp.where(anyv, bmv, jnp.float32(0.0)),
                               mask=lane0)
        return carry

    lax.fori_loop(0, _TOPK, round_body, 0)

    @pl.when(w == 0)
    def _():
        ooff = pl.multiple_of(b * _OUTPAD, 8)
        pltpu.sync_copy(o1v, ox1_hbm.at[pl.ds(ooff, _OUTPAD)])
        pltpu.sync_copy(o2v, oy1_hbm.at[pl.ds(ooff, _OUTPAD)])
        pltpu.sync_copy(o3v, ox2_hbm.at[pl.ds(ooff, _OUTPAD)])
        pltpu.sync_copy(o4v, oy2_hbm.at[pl.ds(ooff, _OUTPAD)])
        pltpu.sync_copy(olv, olab_hbm.at[pl.ds(ooff, _OUTPAD)])
        pltpu.sync_copy(osv, osc_hbm.at[pl.ds(ooff, _OUTPAD)])


def _run_nms_sc(boxes_t, scores, labels):
    mesh = plsc.VectorSubcoreMesh(core_axis_name="c", subcore_axis_name="s")
    f32 = jnp.float32
    i32 = jnp.int32
    kern = pl.kernel(
        _nms_sc,
        mesh=mesh,
        compiler_params=pltpu.CompilerParams(needs_layout_passes=False),
        out_type=(
            jax.ShapeDtypeStruct((_B * _OUTPAD,), f32),
            jax.ShapeDtypeStruct((_B * _OUTPAD,), f32),
            jax.ShapeDtypeStruct((_B * _OUTPAD,), f32),
            jax.ShapeDtypeStruct((_B * _OUTPAD,), f32),
            jax.ShapeDtypeStruct((_B * _OUTPAD,), i32),
            jax.ShapeDtypeStruct((_B * _OUTPAD,), f32),
        ),
        scratch_types=[
            pltpu.VMEM((_NPAD,), f32),
            pltpu.VMEM((_NPAD,), f32),
            pltpu.VMEM((_NPAD,), f32),
            pltpu.VMEM((_NPAD,), f32),
            pltpu.VMEM((_NPAD,), i32),
            pltpu.VMEM((_PER_W,), f32),
            pltpu.VMEM((16,), f32),
            pltpu.VMEM((16,), i32),
            pltpu.VMEM((64,), f32),
            pltpu.VMEM((64,), i32),
            pltpu.VMEM((_OUTPAD,), f32),
            pltpu.VMEM((_OUTPAD,), f32),
            pltpu.VMEM((_OUTPAD,), f32),
            pltpu.VMEM((_OUTPAD,), f32),
            pltpu.VMEM((_OUTPAD,), i32),
            pltpu.VMEM((_OUTPAD,), f32),
            pltpu.VMEM_SHARED((512,), f32),
            pltpu.VMEM_SHARED((512,), i32),
        ],
    )
    return kern(boxes_t.reshape(-1), scores.reshape(-1), labels.reshape(-1))


def kernel(x, params):
    loc, cls = _ssd_forward(x, params)
    B, N, _ = loc.shape

    boxes_t = jnp.transpose(loc, (2, 0, 1))
    boxes_t = jnp.pad(boxes_t, ((0, 0), (0, 0), (0, _NPAD - N)))
    cls_t = jnp.transpose(cls, (2, 0, 1))
    cls_t = jnp.pad(cls_t, ((0, 0), (0, 0), (0, _NPAD - N)),
                    constant_values=-jnp.inf)

    scores, labels = pl.pallas_call(
        _score_kernel,
        out_shape=(
            jax.ShapeDtypeStruct((B, _NPAD), jnp.float32),
            jax.ShapeDtypeStruct((B, _NPAD), jnp.int32),
        ),
    )(cls_t)

    outs = _run_nms_sc(boxes_t, scores, labels)
    ox1, oy1, ox2, oy2, olab, osc = (o.reshape(_B, _OUTPAD) for o in outs)

    boxes_out = jnp.stack(
        [ox1[:, :_TOPK], oy1[:, :_TOPK], ox2[:, :_TOPK], oy2[:, :_TOPK]],
        axis=-1)
    return boxes_out, olab[:, :_TOPK], osc[:, :_TOPK]


# speedup vs baseline: 1.0237x; 1.0237x over previous
import functools
import math

import jax
import jax.numpy as jnp
from jax import lax
from jax.experimental import pallas as pl
from jax.experimental.pallas import tpu as pltpu
from jax.experimental.pallas import tpu_sc as plsc

_CFGS = [[1, 16, 1, 1], [6, 24, 2, 2], [6, 32, 3, 2], [6, 64, 4, 2],
         [6, 96, 3, 1], [6, 160, 3, 2], [6, 320, 1, 1]]
_NUM_CLASSES = 21
_TOPK = 200
_IOU_THR = 0.5

_B = 8
_NPAD = 2816
_PER_W = 704
_CHUNKS = 44
_OUTPAD = 208
_WPI = 4
_BIG = 2 ** 30


def _make_specs():
    specs = [('conv', 3, 32, 3, 2, 1, 1)]
    in_ch = 32
    for t, c, n, s in _CFGS:
        for i in range(n):
            stride = s if i == 0 else 1
            specs.append(('ir', in_ch, c, stride, t))
            in_ch = c
    specs.append(('conv', in_ch, 1280, 1, 1, 0, 1))
    return specs


def _conv2d(x, w, stride, padding, groups=1):
    return jax.lax.conv_general_dilated(
        x, w, (stride, stride), [(padding, padding), (padding, padding)],
        dimension_numbers=('NCHW', 'OIHW', 'NCHW'), feature_group_count=groups)


def _bn(x, g, b, eps=1e-5):
    mean = jnp.mean(x, axis=(0, 2, 3), keepdims=True)
    var = jnp.var(x, axis=(0, 2, 3), keepdims=True)
    xn = (x - mean) / jnp.sqrt(var + eps)
    return xn * g[None, :, None, None] + b[None, :, None, None]


def _relu6(x):
    return jnp.clip(x, 0.0, 6.0)


def _ssd_forward(x, params):
    specs = _make_specs()
    feats = []
    for li, spec in enumerate(specs):
        if spec[0] == 'conv':
            _, ci, co, ks, st, pd, gr = spec
            x = _relu6(_bn(_conv2d(x, params[f'l{li}_w'], st, pd, gr),
                           params[f'l{li}_g'], params[f'l{li}_b']))
        else:
            _, ci, co, st, t = spec
            hid = ci * t
            h = x
            if t != 1:
                h = _relu6(_bn(_conv2d(h, params[f'l{li}_pw1_w'], 1, 0),
                               params[f'l{li}_pw1_g'], params[f'l{li}_pw1_b']))
            h = _relu6(_bn(_conv2d(h, params[f'l{li}_dw_w'], st, 1, groups=hid),
                           params[f'l{li}_dw_g'], params[f'l{li}_dw_b']))
            h = _bn(_conv2d(h, params[f'l{li}_pw2_w'], 1, 0),
                    params[f'l{li}_pw2_g'], params[f'l{li}_pw2_b'])
            if st == 1 and ci == co:
                h = x + h
            x = h
        if li == 13 or li == 17:
            feats.append(x)
    loc_list, cls_list = [], []
    for i, f in enumerate(feats):
        lp = _conv2d(f, params[f'loc{i}_w'], 1, 1) + params[f'loc{i}_b2'][None, :, None, None]
        cp = _conv2d(f, params[f'cls{i}_w'], 1, 1) + params[f'cls{i}_b2'][None, :, None, None]
        loc_list.append(jnp.transpose(lp, (0, 2, 3, 1)).reshape(lp.shape[0], -1))
        cls_list.append(jnp.transpose(cp, (0, 2, 3, 1)).reshape(cp.shape[0], -1))
    B = x.shape[0]
    loc = jnp.concatenate(loc_list, axis=1).reshape(B, -1, 4)
    cls = jnp.concatenate(cls_list, axis=1).reshape(B, -1, _NUM_CLASSES)
    return loc, cls


def _score_kernel(cls_ref, scores_ref, labels_ref):
    C = cls_ref.shape[0]
    scores = cls_ref[0]
    labels = jnp.zeros(scores.shape, jnp.int32)
    for k in range(1, C):
        cur = cls_ref[k]
        better = cur > scores
        scores = jnp.where(better, cur, scores)
        labels = jnp.where(better, k, labels)
    scores_ref[...] = scores
    labels_ref[...] = labels


def _nms_sc(boxes_hbm, scores_hbm, labels_hbm,
            ox1_hbm, oy1_hbm, ox2_hbm, oy2_hbm, olab_hbm, osc_hbm,
            x1v, y1v, x2v, y2v, labv, sv, areav,
            pairv, mrgv,
            o1v, o2v, o3v, o4v, olv, osv,
            stage):
    neg_inf = jnp.float32(-jnp.inf)
    c = lax.axis_index("c")
    s = lax.axis_index("s")
    b = c * (_B // 2) + s // _WPI
    w = s % _WPI
    base = w * _PER_W
    lanes = lax.iota(jnp.int32, 16)
    lane0 = lanes == 0
    s0 = (s // _WPI) * _WPI

    boff = pl.multiple_of(b * _NPAD, 8)
    pltpu.sync_copy(boxes_hbm.at[pl.ds(pl.multiple_of((0 * _B + b) * _NPAD, 8), _NPAD)], x1v)
    pltpu.sync_copy(boxes_hbm.at[pl.ds(pl.multiple_of((1 * _B + b) * _NPAD, 8), _NPAD)], y1v)
    pltpu.sync_copy(boxes_hbm.at[pl.ds(pl.multiple_of((2 * _B + b) * _NPAD, 8), _NPAD)], x2v)
    pltpu.sync_copy(boxes_hbm.at[pl.ds(pl.multiple_of((3 * _B + b) * _NPAD, 8), _NPAD)], y2v)
    pltpu.sync_copy(labels_hbm.at[pl.ds(boff, _NPAD)], labv)
    pltpu.sync_copy(scores_hbm.at[pl.ds(pl.multiple_of(b * _NPAD + base, 8), _PER_W)], sv)

    def mkarea(cix, _):
        off = base + cix * 16
        ar = ((x2v[pl.ds(off, 16)] - x1v[pl.ds(off, 16)])
              * (y2v[pl.ds(off, 16)] - y1v[pl.ds(off, 16)]))
        areav[pl.ds(cix * 16, 16)] = ar
        return 0
    lax.fori_loop(0, _CHUNKS, mkarea, 0, unroll=4)

    def amax(cix, car):
        bm, bi = car
        v = sv[pl.ds(cix * 16, 16)]
        idx = base + cix * 16 + lanes
        upd = v > bm
        return (jnp.where(upd, v, bm), jnp.where(upd, idx, bi))
    bm0, bi0 = lax.fori_loop(
        0, _CHUNKS, amax,
        (jnp.full((16,), neg_inf, jnp.float32),
         jnp.full((16,), _BIG, jnp.int32)), unroll=4)

    def round_body(t, carry):
        bm, bi = carry
        for sh in (8, 4, 2, 1):
            part = lanes ^ sh
            m2 = bm.at[part].get(mode='promise_in_bounds', unique_indices=True)
            g2 = bi.at[part].get(mode='promise_in_bounds', unique_indices=True)
            take = (m2 > bm) | ((m2 == bm) & (g2 < bi))
            bm = jnp.where(take, m2, bm)
            bi = jnp.where(take, g2, bi)

        p = t % 2
        pairv[pl.ds(0, 16)] = bm
        pairv[pl.ds(16, 16)] = plsc.bitcast(bi, jnp.float32)
        woff = pl.multiple_of(p * 512 + s * 32, 8)
        roff = pl.multiple_of(p * 512 + s0 * 32, 8)
        pltpu.sync_copy(pairv, stage.at[pl.ds(woff, 32)])
        plsc.subcore_barrier()
        pltpu.sync_copy(stage.at[pl.ds(roff, 4 * 32)], mrgv)

        bmv = mrgv[pl.ds(0, 16)]
        bgv = plsc.bitcast(mrgv[pl.ds(16, 16)], jnp.int32)
        for k in range(1, _WPI):
            mk = mrgv[pl.ds(k * 32, 16)]
            gk = plsc.bitcast(mrgv[pl.ds(k * 32 + 16, 16)], jnp.int32)
            take = (mk > bmv) | ((mk == bmv) & (gk < bgv))
            bmv = jnp.where(take, mk, bmv)
            bgv = jnp.where(take, gk, bgv)
        anyv = bmv != neg_inf
        giv = jnp.where(anyv, bgv, 0)

        cx1 = plsc.load_gather(x1v, [giv])
        cy1 = plsc.load_gather(y1v, [giv])
        cx2 = plsc.load_gather(x2v, [giv])
        cy2 = plsc.load_gather(y2v, [giv])
        car = (cx2 - cx1) * (cy2 - cy1)

        def sup(cix, car2):
            nbm, nbi = car2
            off = base + cix * 16
            vx1 = x1v[pl.ds(off, 16)]
            vy1 = y1v[pl.ds(off, 16)]
            vx2 = x2v[pl.ds(off, 16)]
            vy2 = y2v[pl.ds(off, 16)]
            ss = sv[pl.ds(cix * 16, 16)]
            xx1 = jnp.maximum(cx1, vx1)
            yy1 = jnp.maximum(cy1, vy1)
            xx2 = jnp.minimum(cx2, vx2)
            yy2 = jnp.minimum(cy2, vy2)
            wd = jnp.maximum(xx2 - xx1, 0.0)
            ht = jnp.maximum(yy2 - yy1, 0.0)
            inter = wd * ht
            union = car + areav[pl.ds(cix * 16, 16)] - inter
            iou = inter / union
            idx = off + lanes
            keep = (iou <= _IOU_THR) & (idx != giv)
            ns = jnp.where(keep, ss, neg_inf)
            sv[pl.ds(cix * 16, 16)] = ns
            upd = ns > nbm
            return (jnp.where(upd, ns, nbm), jnp.where(upd, idx, nbi))
        nbm, nbi = lax.fori_loop(
            0, _CHUNKS, sup,
            (jnp.full((16,), neg_inf, jnp.float32),
             jnp.full((16,), _BIG, jnp.int32)), unroll=4)

        @pl.when(w == 0)
        def _():
            mm = jnp.where(anyv, jnp.float32(1.0), jnp.float32(0.0))
            mi = jnp.where(anyv, 1, 0)
            clab = plsc.load_gather(labv, [giv])
            tv = jnp.full((16,), t, jnp.int32)
            plsc.store_scatter(o1v, [tv], cx1 * mm, mask=lane0)
            plsc.store_scatter(o2v, [tv], cy1 * mm, mask=lane0)
            plsc.store_scatter(o3v, [tv], cx2 * mm, mask=lane0)
            plsc.store_scatter(o4v, [tv], cy2 * mm, mask=lane0)
            plsc.store_scatter(olv, [tv], clab * mi, mask=lane0)
            plsc.store_scatter(osv, [tv], jnp.where(anyv, bmv, jnp.float32(0.0)),
                               mask=lane0)
        return (nbm, nbi)

    lax.fori_loop(0, _TOPK, round_body, (bm0, bi0))

    @pl.when(w == 0)
    def _():
        ooff = pl.multiple_of(b * _OUTPAD, 8)
        pltpu.sync_copy(o1v, ox1_hbm.at[pl.ds(ooff, _OUTPAD)])
        pltpu.sync_copy(o2v, oy1_hbm.at[pl.ds(ooff, _OUTPAD)])
        pltpu.sync_copy(o3v, ox2_hbm.at[pl.ds(ooff, _OUTPAD)])
        pltpu.sync_copy(o4v, oy2_hbm.at[pl.ds(ooff, _OUTPAD)])
        pltpu.sync_copy(olv, olab_hbm.at[pl.ds(ooff, _OUTPAD)])
        pltpu.sync_copy(osv, osc_hbm.at[pl.ds(ooff, _OUTPAD)])


def _run_nms_sc(boxes_t, scores, labels):
    mesh = plsc.VectorSubcoreMesh(core_axis_name="c", subcore_axis_name="s")
    f32 = jnp.float32
    i32 = jnp.int32
    kern = pl.kernel(
        _nms_sc,
        mesh=mesh,
        compiler_params=pltpu.CompilerParams(needs_layout_passes=False),
        out_type=(
            jax.ShapeDtypeStruct((_B * _OUTPAD,), f32),
            jax.ShapeDtypeStruct((_B * _OUTPAD,), f32),
            jax.ShapeDtypeStruct((_B * _OUTPAD,), f32),
            jax.ShapeDtypeStruct((_B * _OUTPAD,), f32),
            jax.ShapeDtypeStruct((_B * _OUTPAD,), i32),
            jax.ShapeDtypeStruct((_B * _OUTPAD,), f32),
        ),
        scratch_types=[
            pltpu.VMEM((_NPAD,), f32),
            pltpu.VMEM((_NPAD,), f32),
            pltpu.VMEM((_NPAD,), f32),
            pltpu.VMEM((_NPAD,), f32),
            pltpu.VMEM((_NPAD,), i32),
            pltpu.VMEM((_PER_W,), f32),
            pltpu.VMEM((_PER_W,), f32),
            pltpu.VMEM((32,), f32),
            pltpu.VMEM((128,), f32),
            pltpu.VMEM((_OUTPAD,), f32),
            pltpu.VMEM((_OUTPAD,), f32),
            pltpu.VMEM((_OUTPAD,), f32),
            pltpu.VMEM((_OUTPAD,), f32),
            pltpu.VMEM((_OUTPAD,), i32),
            pltpu.VMEM((_OUTPAD,), f32),
            pltpu.VMEM_SHARED((1024,), f32),
        ],
    )
    return kern(boxes_t.reshape(-1), scores.reshape(-1), labels.reshape(-1))


def kernel(x, params):
    loc, cls = _ssd_forward(x, params)
    B, N, _ = loc.shape

    boxes_t = jnp.transpose(loc, (2, 0, 1))
    boxes_t = jnp.pad(boxes_t, ((0, 0), (0, 0), (0, _NPAD - N)))
    cls_t = jnp.transpose(cls, (2, 0, 1))
    cls_t = jnp.pad(cls_t, ((0, 0), (0, 0), (0, _NPAD - N)),
                    constant_values=-jnp.inf)

    scores, labels = pl.pallas_call(
        _score_kernel,
        out_shape=(
            jax.ShapeDtypeStruct((B, _NPAD), jnp.float32),
            jax.ShapeDtypeStruct((B, _NPAD), jnp.int32),
        ),
    )(cls_t)

    outs = _run_nms_sc(boxes_t, scores, labels)
    ox1, oy1, ox2, oy2, olab, osc = (o.reshape(_B, _OUTPAD) for o in outs)

    boxes_out = jnp.stack(
        [ox1[:, :_TOPK], oy1[:, :_TOPK], ox2[:, :_TOPK], oy2[:, :_TOPK]],
        axis=-1)
    return boxes_out, olab[:, :_TOPK], osc[:, :_TOPK]


# speedup vs baseline: 1.0267x; 1.0029x over previous
import functools
import math

import jax
import jax.numpy as jnp
from jax import lax
from jax.experimental import pallas as pl
from jax.experimental.pallas import tpu as pltpu
from jax.experimental.pallas import tpu_sc as plsc

_CFGS = [[1, 16, 1, 1], [6, 24, 2, 2], [6, 32, 3, 2], [6, 64, 4, 2],
         [6, 96, 3, 1], [6, 160, 3, 2], [6, 320, 1, 1]]
_NUM_CLASSES = 21
_TOPK = 200
_IOU_THR = 0.5

_B = 8
_NPAD = 2816
_PER_W = 704
_CHUNKS = 44
_OUTPAD = 208
_WPI = 4
_BIG = 2 ** 30


def _make_specs():
    specs = [('conv', 3, 32, 3, 2, 1, 1)]
    in_ch = 32
    for t, c, n, s in _CFGS:
        for i in range(n):
            stride = s if i == 0 else 1
            specs.append(('ir', in_ch, c, stride, t))
            in_ch = c
    specs.append(('conv', in_ch, 1280, 1, 1, 0, 1))
    return specs


def _conv2d(x, w, stride, padding, groups=1):
    return jax.lax.conv_general_dilated(
        x, w, (stride, stride), [(padding, padding), (padding, padding)],
        dimension_numbers=('NCHW', 'OIHW', 'NCHW'), feature_group_count=groups)


def _bn(x, g, b, eps=1e-5):
    mean = jnp.mean(x, axis=(0, 2, 3), keepdims=True)
    var = jnp.var(x, axis=(0, 2, 3), keepdims=True)
    xn = (x - mean) / jnp.sqrt(var + eps)
    return xn * g[None, :, None, None] + b[None, :, None, None]


def _relu6(x):
    return jnp.clip(x, 0.0, 6.0)


def _ssd_forward(x, params):
    specs = _make_specs()
    feats = []
    for li, spec in enumerate(specs):
        if spec[0] == 'conv':
            _, ci, co, ks, st, pd, gr = spec
            x = _relu6(_bn(_conv2d(x, params[f'l{li}_w'], st, pd, gr),
                           params[f'l{li}_g'], params[f'l{li}_b']))
        else:
            _, ci, co, st, t = spec
            hid = ci * t
            h = x
            if t != 1:
                h = _relu6(_bn(_conv2d(h, params[f'l{li}_pw1_w'], 1, 0),
                               params[f'l{li}_pw1_g'], params[f'l{li}_pw1_b']))
            h = _relu6(_bn(_conv2d(h, params[f'l{li}_dw_w'], st, 1, groups=hid),
                           params[f'l{li}_dw_g'], params[f'l{li}_dw_b']))
            h = _bn(_conv2d(h, params[f'l{li}_pw2_w'], 1, 0),
                    params[f'l{li}_pw2_g'], params[f'l{li}_pw2_b'])
            if st == 1 and ci == co:
                h = x + h
            x = h
        if li == 13 or li == 17:
            feats.append(x)
    loc_list, cls_list = [], []
    for i, f in enumerate(feats):
        lp = _conv2d(f, params[f'loc{i}_w'], 1, 1) + params[f'loc{i}_b2'][None, :, None, None]
        cp = _conv2d(f, params[f'cls{i}_w'], 1, 1) + params[f'cls{i}_b2'][None, :, None, None]
        loc_list.append(jnp.transpose(lp, (0, 2, 3, 1)).reshape(lp.shape[0], -1))
        cls_list.append(jnp.transpose(cp, (0, 2, 3, 1)).reshape(cp.shape[0], -1))
    B = x.shape[0]
    loc = jnp.concatenate(loc_list, axis=1).reshape(B, -1, 4)
    cls = jnp.concatenate(cls_list, axis=1).reshape(B, -1, _NUM_CLASSES)
    return loc, cls


def _score_kernel(cls_ref, scores_ref, labels_ref):
    C = cls_ref.shape[0]
    scores = cls_ref[0]
    labels = jnp.zeros(scores.shape, jnp.int32)
    for k in range(1, C):
        cur = cls_ref[k]
        better = cur > scores
        scores = jnp.where(better, cur, scores)
        labels = jnp.where(better, k, labels)
    scores_ref[...] = scores
    labels_ref[...] = labels


def _nms_sc(boxes_hbm, scores_hbm, labels_hbm,
            ox1_hbm, oy1_hbm, ox2_hbm, oy2_hbm, olab_hbm, osc_hbm,
            x1v, y1v, x2v, y2v, labv, sv, areav,
            pairv, mrgv,
            o1v, o2v, o3v, o4v, olv, osv,
            stage):
    neg_inf = jnp.float32(-jnp.inf)
    c = lax.axis_index("c")
    s = lax.axis_index("s")
    b = c * (_B // 2) + s // _WPI
    w = s % _WPI
    base = w * _PER_W
    lanes = lax.iota(jnp.int32, 16)
    lane0 = lanes == 0
    s0 = (s // _WPI) * _WPI

    boff = pl.multiple_of(b * _NPAD, 8)
    pltpu.sync_copy(boxes_hbm.at[pl.ds(pl.multiple_of((0 * _B + b) * _NPAD, 8), _NPAD)], x1v)
    pltpu.sync_copy(boxes_hbm.at[pl.ds(pl.multiple_of((1 * _B + b) * _NPAD, 8), _NPAD)], y1v)
    pltpu.sync_copy(boxes_hbm.at[pl.ds(pl.multiple_of((2 * _B + b) * _NPAD, 8), _NPAD)], x2v)
    pltpu.sync_copy(boxes_hbm.at[pl.ds(pl.multiple_of((3 * _B + b) * _NPAD, 8), _NPAD)], y2v)
    pltpu.sync_copy(labels_hbm.at[pl.ds(boff, _NPAD)], labv)
    pltpu.sync_copy(scores_hbm.at[pl.ds(pl.multiple_of(b * _NPAD + base, 8), _PER_W)], sv)

    def mkarea(cix, _):
        off = base + cix * 16
        ar = ((x2v[pl.ds(off, 16)] - x1v[pl.ds(off, 16)])
              * (y2v[pl.ds(off, 16)] - y1v[pl.ds(off, 16)]))
        areav[pl.ds(cix * 16, 16)] = ar
        return 0
    lax.fori_loop(0, _CHUNKS, mkarea, 0, unroll=4)

    def amax(cix, car):
        bm, bi = car
        v = sv[pl.ds(cix * 16, 16)]
        idx = base + cix * 16 + lanes
        upd = v > bm
        return (jnp.where(upd, v, bm), jnp.where(upd, idx, bi))
    bm0, bi0 = lax.fori_loop(
        0, _CHUNKS, amax,
        (jnp.full((16,), neg_inf, jnp.float32),
         jnp.full((16,), _BIG, jnp.int32)), unroll=4)

    def round_body(t, carry):
        bm, bi = carry
        for sh in (8, 4, 2, 1):
            part = lanes ^ sh
            m2 = bm.at[part].get(mode='promise_in_bounds', unique_indices=True)
            g2 = bi.at[part].get(mode='promise_in_bounds', unique_indices=True)
            take = (m2 > bm) | ((m2 == bm) & (g2 < bi))
            bm = jnp.where(take, m2, bm)
            bi = jnp.where(take, g2, bi)

        p = t % 2
        pairv[pl.ds(0, 16)] = bm
        pairv[pl.ds(16, 16)] = plsc.bitcast(bi, jnp.float32)
        woff = pl.multiple_of(p * 512 + s * 32, 8)
        roff = pl.multiple_of(p * 512 + s0 * 32, 8)
        pltpu.sync_copy(pairv, stage.at[pl.ds(woff, 32)])
        plsc.subcore_barrier()
        pltpu.sync_copy(stage.at[pl.ds(roff, 4 * 32)], mrgv)

        bmv = mrgv[pl.ds(0, 16)]
        bgv = plsc.bitcast(mrgv[pl.ds(16, 16)], jnp.int32)
        for k in range(1, _WPI):
            mk = mrgv[pl.ds(k * 32, 16)]
            gk = plsc.bitcast(mrgv[pl.ds(k * 32 + 16, 16)], jnp.int32)
            take = (mk > bmv) | ((mk == bmv) & (gk < bgv))
            bmv = jnp.where(take, mk, bmv)
            bgv = jnp.where(take, gk, bgv)
        anyv = bmv != neg_inf
        giv = jnp.where(anyv, bgv, 0)

        cx1 = plsc.load_gather(x1v, [giv])
        cy1 = plsc.load_gather(y1v, [giv])
        cx2 = plsc.load_gather(x2v, [giv])
        cy2 = plsc.load_gather(y2v, [giv])
        car = (cx2 - cx1) * (cy2 - cy1)

        liv = jnp.clip(giv - base, 0, _PER_W - 1)
        own = anyv & (giv >= base) & (giv < base + _PER_W)
        plsc.store_scatter(sv, [liv], jnp.full((16,), neg_inf, jnp.float32),
                           mask=lane0 & own)

        def sup(cix, car2):
            nbm, nbi = car2
            off = base + cix * 16
            vx1 = x1v[pl.ds(off, 16)]
            vy1 = y1v[pl.ds(off, 16)]
            vx2 = x2v[pl.ds(off, 16)]
            vy2 = y2v[pl.ds(off, 16)]
            ss = sv[pl.ds(cix * 16, 16)]
            xx1 = jnp.maximum(cx1, vx1)
            yy1 = jnp.maximum(cy1, vy1)
            xx2 = jnp.minimum(cx2, vx2)
            yy2 = jnp.minimum(cy2, vy2)
            wd = jnp.maximum(xx2 - xx1, 0.0)
            ht = jnp.maximum(yy2 - yy1, 0.0)
            inter = wd * ht
            union = car + areav[pl.ds(cix * 16, 16)] - inter
            iou = inter / union
            keep = iou <= _IOU_THR
            ns = jnp.where(keep, ss, neg_inf)
            sv[pl.ds(cix * 16, 16)] = ns
            upd = ns > nbm
            idx = off + lanes
            return (jnp.where(upd, ns, nbm), jnp.where(upd, idx, nbi))
        nbm, nbi = lax.fori_loop(
            0, _CHUNKS, sup,
            (jnp.full((16,), neg_inf, jnp.float32),
             jnp.full((16,), _BIG, jnp.int32)), unroll=11)

        @pl.when(w == 0)
        def _():
            mm = jnp.where(anyv, jnp.float32(1.0), jnp.float32(0.0))
            mi = jnp.where(anyv, 1, 0)
            clab = plsc.load_gather(labv, [giv])
            tv = jnp.full((16,), t, jnp.int32)
            plsc.store_scatter(o1v, [tv], cx1 * mm, mask=lane0)
            plsc.store_scatter(o2v, [tv], cy1 * mm, mask=lane0)
            plsc.store_scatter(o3v, [tv], cx2 * mm, mask=lane0)
            plsc.store_scatter(o4v, [tv], cy2 * mm, mask=lane0)
            plsc.store_scatter(olv, [tv], clab * mi, mask=lane0)
            plsc.store_scatter(osv, [tv], jnp.where(anyv, bmv, jnp.float32(0.0)),
                               mask=lane0)
        return (nbm, nbi)

    lax.fori_loop(0, _TOPK, round_body, (bm0, bi0))

    @pl.when(w == 0)
    def _():
        ooff = pl.multiple_of(b * _OUTPAD, 8)
        pltpu.sync_copy(o1v, ox1_hbm.at[pl.ds(ooff, _OUTPAD)])
        pltpu.sync_copy(o2v, oy1_hbm.at[pl.ds(ooff, _OUTPAD)])
        pltpu.sync_copy(o3v, ox2_hbm.at[pl.ds(ooff, _OUTPAD)])
        pltpu.sync_copy(o4v, oy2_hbm.at[pl.ds(ooff, _OUTPAD)])
        pltpu.sync_copy(olv, olab_hbm.at[pl.ds(ooff, _OUTPAD)])
        pltpu.sync_copy(osv, osc_hbm.at[pl.ds(ooff, _OUTPAD)])


def _run_nms_sc(boxes_t, scores, labels):
    mesh = plsc.VectorSubcoreMesh(core_axis_name="c", subcore_axis_name="s")
    f32 = jnp.float32
    i32 = jnp.int32
    kern = pl.kernel(
        _nms_sc,
        mesh=mesh,
        compiler_params=pltpu.CompilerParams(needs_layout_passes=False),
        out_type=(
            jax.ShapeDtypeStruct((_B * _OUTPAD,), f32),
            jax.ShapeDtypeStruct((_B * _OUTPAD,), f32),
            jax.ShapeDtypeStruct((_B * _OUTPAD,), f32),
            jax.ShapeDtypeStruct((_B * _OUTPAD,), f32),
            jax.ShapeDtypeStruct((_B * _OUTPAD,), i32),
            jax.ShapeDtypeStruct((_B * _OUTPAD,), f32),
        ),
        scratch_types=[
            pltpu.VMEM((_NPAD,), f32),
            pltpu.VMEM((_NPAD,), f32),
            pltpu.VMEM((_NPAD,), f32),
            pltpu.VMEM((_NPAD,), f32),
            pltpu.VMEM((_NPAD,), i32),
            pltpu.VMEM((_PER_W,), f32),
            pltpu.VMEM((_PER_W,), f32),
            pltpu.VMEM((32,), f32),
            pltpu.VMEM((128,), f32),
            pltpu.VMEM((_OUTPAD,), f32),
            pltpu.VMEM((_OUTPAD,), f32),
            pltpu.VMEM((_OUTPAD,), f32),
            pltpu.VMEM((_OUTPAD,), f32),
            pltpu.VMEM((_OUTPAD,), i32),
            pltpu.VMEM((_OUTPAD,), f32),
            pltpu.VMEM_SHARED((1024,), f32),
        ],
    )
    return kern(boxes_t.reshape(-1), scores.reshape(-1), labels.reshape(-1))


def kernel(x, params):
    loc, cls = _ssd_forward(x, params)
    B, N, _ = loc.shape

    boxes_t = jnp.transpose(loc, (2, 0, 1))
    boxes_t = jnp.pad(boxes_t, ((0, 0), (0, 0), (0, _NPAD - N)))
    cls_t = jnp.transpose(cls, (2, 0, 1))
    cls_t = jnp.pad(cls_t, ((0, 0), (0, 0), (0, _NPAD - N)),
                    constant_values=-jnp.inf)

    scores, labels = pl.pallas_call(
        _score_kernel,
        out_shape=(
            jax.ShapeDtypeStruct((B, _NPAD), jnp.float32),
            jax.ShapeDtypeStruct((B, _NPAD), jnp.int32),
        ),
    )(cls_t)

    outs = _run_nms_sc(boxes_t, scores, labels)
    ox1, oy1, ox2, oy2, olab, osc = (o.reshape(_B, _OUTPAD) for o in outs)

    boxes_out = jnp.stack(
        [ox1[:, :_TOPK], oy1[:, :_TOPK], ox2[:, :_TOPK], oy2[:, :_TOPK]],
        axis=-1)
    return boxes_out, olab[:, :_TOPK], osc[:, :_TOPK]


# speedup vs baseline: 1.0282x; 1.0015x over previous
import functools
import math

import jax
import jax.numpy as jnp
from jax import lax
from jax.experimental import pallas as pl
from jax.experimental.pallas import tpu as pltpu
from jax.experimental.pallas import tpu_sc as plsc

_CFGS = [[1, 16, 1, 1], [6, 24, 2, 2], [6, 32, 3, 2], [6, 64, 4, 2],
         [6, 96, 3, 1], [6, 160, 3, 2], [6, 320, 1, 1]]
_NUM_CLASSES = 21
_TOPK = 200
_IOU_THR = 0.5

_B = 8
_NPAD = 2816
_PER_W = 704
_CHUNKS = 44
_OUTPAD = 208
_WPI = 4
_BIG = 2 ** 30


def _make_specs():
    specs = [('conv', 3, 32, 3, 2, 1, 1)]
    in_ch = 32
    for t, c, n, s in _CFGS:
        for i in range(n):
            stride = s if i == 0 else 1
            specs.append(('ir', in_ch, c, stride, t))
            in_ch = c
    specs.append(('conv', in_ch, 1280, 1, 1, 0, 1))
    return specs


def _conv2d(x, w, stride, padding, groups=1):
    return jax.lax.conv_general_dilated(
        x, w, (stride, stride), [(padding, padding), (padding, padding)],
        dimension_numbers=('NCHW', 'OIHW', 'NCHW'), feature_group_count=groups)


def _bn(x, g, b, eps=1e-5):
    mean = jnp.mean(x, axis=(0, 2, 3), keepdims=True)
    var = jnp.var(x, axis=(0, 2, 3), keepdims=True)
    xn = (x - mean) / jnp.sqrt(var + eps)
    return xn * g[None, :, None, None] + b[None, :, None, None]


def _relu6(x):
    return jnp.clip(x, 0.0, 6.0)


def _ssd_forward(x, params):
    specs = _make_specs()
    feats = []
    for li, spec in enumerate(specs):
        if spec[0] == 'conv':
            _, ci, co, ks, st, pd, gr = spec
            x = _relu6(_bn(_conv2d(x, params[f'l{li}_w'], st, pd, gr),
                           params[f'l{li}_g'], params[f'l{li}_b']))
        else:
            _, ci, co, st, t = spec
            hid = ci * t
            h = x
            if t != 1:
                h = _relu6(_bn(_conv2d(h, params[f'l{li}_pw1_w'], 1, 0),
                               params[f'l{li}_pw1_g'], params[f'l{li}_pw1_b']))
            h = _relu6(_bn(_conv2d(h, params[f'l{li}_dw_w'], st, 1, groups=hid),
                           params[f'l{li}_dw_g'], params[f'l{li}_dw_b']))
            h = _bn(_conv2d(h, params[f'l{li}_pw2_w'], 1, 0),
                    params[f'l{li}_pw2_g'], params[f'l{li}_pw2_b'])
            if st == 1 and ci == co:
                h = x + h
            x = h
        if li == 13 or li == 17:
            feats.append(x)
    loc_list, cls_list = [], []
    for i, f in enumerate(feats):
        lp = _conv2d(f, params[f'loc{i}_w'], 1, 1) + params[f'loc{i}_b2'][None, :, None, None]
        cp = _conv2d(f, params[f'cls{i}_w'], 1, 1) + params[f'cls{i}_b2'][None, :, None, None]
        loc_list.append(jnp.transpose(lp, (0, 2, 3, 1)).reshape(lp.shape[0], -1))
        cls_list.append(jnp.transpose(cp, (0, 2, 3, 1)).reshape(cp.shape[0], -1))
    B = x.shape[0]
    loc = jnp.concatenate(loc_list, axis=1).reshape(B, -1, 4)
    cls = jnp.concatenate(cls_list, axis=1).reshape(B, -1, _NUM_CLASSES)
    return loc, cls


def _score_kernel(cls_ref, scores_ref, labels_ref):
    C = cls_ref.shape[0]
    scores = cls_ref[0]
    labels = jnp.zeros(scores.shape, jnp.int32)
    for k in range(1, C):
        cur = cls_ref[k]
        better = cur > scores
        scores = jnp.where(better, cur, scores)
        labels = jnp.where(better, k, labels)
    scores_ref[...] = scores
    labels_ref[...] = labels


def _nms_sc(boxes_hbm, scores_hbm, labels_hbm,
            ox1_hbm, oy1_hbm, ox2_hbm, oy2_hbm, olab_hbm, osc_hbm,
            x1v, y1v, x2v, y2v, labv, sv, areav,
            pairv, mrgv,
            o1v, o2v, o3v, o4v, olv, osv,
            stage):
    neg_inf = jnp.float32(-jnp.inf)
    c = lax.axis_index("c")
    s = lax.axis_index("s")
    b = c * (_B // 2) + s // _WPI
    w = s % _WPI
    base = w * _PER_W
    lanes = lax.iota(jnp.int32, 16)
    lane0 = lanes == 0
    s0 = (s // _WPI) * _WPI

    boff = pl.multiple_of(b * _NPAD, 8)
    pltpu.sync_copy(boxes_hbm.at[pl.ds(pl.multiple_of((0 * _B + b) * _NPAD, 8), _NPAD)], x1v)
    pltpu.sync_copy(boxes_hbm.at[pl.ds(pl.multiple_of((1 * _B + b) * _NPAD, 8), _NPAD)], y1v)
    pltpu.sync_copy(boxes_hbm.at[pl.ds(pl.multiple_of((2 * _B + b) * _NPAD, 8), _NPAD)], x2v)
    pltpu.sync_copy(boxes_hbm.at[pl.ds(pl.multiple_of((3 * _B + b) * _NPAD, 8), _NPAD)], y2v)
    pltpu.sync_copy(labels_hbm.at[pl.ds(boff, _NPAD)], labv)
    pltpu.sync_copy(scores_hbm.at[pl.ds(pl.multiple_of(b * _NPAD + base, 8), _PER_W)], sv)

    def mkarea(cix, _):
        off = base + cix * 16
        ar = ((x2v[pl.ds(off, 16)] - x1v[pl.ds(off, 16)])
              * (y2v[pl.ds(off, 16)] - y1v[pl.ds(off, 16)]))
        areav[pl.ds(cix * 16, 16)] = ar
        return 0
    lax.fori_loop(0, _CHUNKS, mkarea, 0, unroll=4)

    def amax(cix, car):
        bm, bi = car
        v = sv[pl.ds(cix * 16, 16)]
        idx = base + cix * 16 + lanes
        upd = v > bm
        return (jnp.where(upd, v, bm), jnp.where(upd, idx, bi))
    bm0, bi0 = lax.fori_loop(
        0, _CHUNKS, amax,
        (jnp.full((16,), neg_inf, jnp.float32),
         jnp.full((16,), _BIG, jnp.int32)), unroll=4)

    def round_body(t, carry):
        bm, bi = carry
        for sh in (8, 4, 2, 1):
            part = lanes ^ sh
            m2 = bm.at[part].get(mode='promise_in_bounds', unique_indices=True)
            g2 = bi.at[part].get(mode='promise_in_bounds', unique_indices=True)
            take = (m2 > bm) | ((m2 == bm) & (g2 < bi))
            bm = jnp.where(take, m2, bm)
            bi = jnp.where(take, g2, bi)

        p = t % 2
        pairv[pl.ds(0, 16)] = bm
        pairv[pl.ds(16, 16)] = plsc.bitcast(bi, jnp.float32)
        woff = pl.multiple_of(p * 512 + s * 32, 8)
        roff = pl.multiple_of(p * 512 + s0 * 32, 8)
        pltpu.sync_copy(pairv, stage.at[pl.ds(woff, 32)])
        plsc.subcore_barrier()
        pltpu.sync_copy(stage.at[pl.ds(roff, 4 * 32)], mrgv)

        bmv = mrgv[pl.ds(0, 16)]
        bgv = plsc.bitcast(mrgv[pl.ds(16, 16)], jnp.int32)
        for k in range(1, _WPI):
            mk = mrgv[pl.ds(k * 32, 16)]
            gk = plsc.bitcast(mrgv[pl.ds(k * 32 + 16, 16)], jnp.int32)
            take = (mk > bmv) | ((mk == bmv) & (gk < bgv))
            bmv = jnp.where(take, mk, bmv)
            bgv = jnp.where(take, gk, bgv)
        anyv = bmv != neg_inf
        giv = jnp.where(anyv, bgv, 0)

        cx1 = plsc.load_gather(x1v, [giv])
        cy1 = plsc.load_gather(y1v, [giv])
        cx2 = plsc.load_gather(x2v, [giv])
        cy2 = plsc.load_gather(y2v, [giv])
        car = (cx2 - cx1) * (cy2 - cy1)

        liv = jnp.clip(giv - base, 0, _PER_W - 1)
        own = anyv & (giv >= base) & (giv < base + _PER_W)
        plsc.store_scatter(sv, [liv], jnp.full((16,), neg_inf, jnp.float32),
                           mask=lane0 & own)

        def sup(cix, car2):
            nbm, nbi = car2
            off = base + cix * 16
            vx1 = x1v[pl.ds(off, 16)]
            vy1 = y1v[pl.ds(off, 16)]
            vx2 = x2v[pl.ds(off, 16)]
            vy2 = y2v[pl.ds(off, 16)]
            ss = sv[pl.ds(cix * 16, 16)]
            xx1 = jnp.maximum(cx1, vx1)
            yy1 = jnp.maximum(cy1, vy1)
            xx2 = jnp.minimum(cx2, vx2)
            yy2 = jnp.minimum(cy2, vy2)
            wd = jnp.maximum(xx2 - xx1, 0.0)
            ht = jnp.maximum(yy2 - yy1, 0.0)
            inter = wd * ht
            union = car + areav[pl.ds(cix * 16, 16)] - inter
            iou = inter / union
            keep = iou <= _IOU_THR
            ns = jnp.where(keep, ss, neg_inf)
            sv[pl.ds(cix * 16, 16)] = ns
            upd = ns > nbm
            idx = off + lanes
            return (jnp.where(upd, ns, nbm), jnp.where(upd, idx, nbi))
        nbm, nbi = lax.fori_loop(
            0, _CHUNKS, sup,
            (jnp.full((16,), neg_inf, jnp.float32),
             jnp.full((16,), _BIG, jnp.int32)), unroll=22)

        @pl.when(w == 0)
        def _():
            mm = jnp.where(anyv, jnp.float32(1.0), jnp.float32(0.0))
            mi = jnp.where(anyv, 1, 0)
            clab = plsc.load_gather(labv, [giv])
            tv = jnp.full((16,), t, jnp.int32)
            plsc.store_scatter(o1v, [tv], cx1 * mm, mask=lane0)
            plsc.store_scatter(o2v, [tv], cy1 * mm, mask=lane0)
            plsc.store_scatter(o3v, [tv], cx2 * mm, mask=lane0)
            plsc.store_scatter(o4v, [tv], cy2 * mm, mask=lane0)
            plsc.store_scatter(olv, [tv], clab * mi, mask=lane0)
            plsc.store_scatter(osv, [tv], jnp.where(anyv, bmv, jnp.float32(0.0)),
                               mask=lane0)
        return (nbm, nbi)

    lax.fori_loop(0, _TOPK, round_body, (bm0, bi0))

    @pl.when(w == 0)
    def _():
        ooff = pl.multiple_of(b * _OUTPAD, 8)
        pltpu.sync_copy(o1v, ox1_hbm.at[pl.ds(ooff, _OUTPAD)])
        pltpu.sync_copy(o2v, oy1_hbm.at[pl.ds(ooff, _OUTPAD)])
        pltpu.sync_copy(o3v, ox2_hbm.at[pl.ds(ooff, _OUTPAD)])
        pltpu.sync_copy(o4v, oy2_hbm.at[pl.ds(ooff, _OUTPAD)])
        pltpu.sync_copy(olv, olab_hbm.at[pl.ds(ooff, _OUTPAD)])
        pltpu.sync_copy(osv, osc_hbm.at[pl.ds(ooff, _OUTPAD)])


def _run_nms_sc(boxes_t, scores, labels):
    mesh = plsc.VectorSubcoreMesh(core_axis_name="c", subcore_axis_name="s")
    f32 = jnp.float32
    i32 = jnp.int32
    kern = pl.kernel(
        _nms_sc,
        mesh=mesh,
        compiler_params=pltpu.CompilerParams(needs_layout_passes=False),
        out_type=(
            jax.ShapeDtypeStruct((_B * _OUTPAD,), f32),
            jax.ShapeDtypeStruct((_B * _OUTPAD,), f32),
            jax.ShapeDtypeStruct((_B * _OUTPAD,), f32),
            jax.ShapeDtypeStruct((_B * _OUTPAD,), f32),
            jax.ShapeDtypeStruct((_B * _OUTPAD,), i32),
            jax.ShapeDtypeStruct((_B * _OUTPAD,), f32),
        ),
        scratch_types=[
            pltpu.VMEM((_NPAD,), f32),
            pltpu.VMEM((_NPAD,), f32),
            pltpu.VMEM((_NPAD,), f32),
            pltpu.VMEM((_NPAD,), f32),
            pltpu.VMEM((_NPAD,), i32),
            pltpu.VMEM((_PER_W,), f32),
            pltpu.VMEM((_PER_W,), f32),
            pltpu.VMEM((32,), f32),
            pltpu.VMEM((128,), f32),
            pltpu.VMEM((_OUTPAD,), f32),
            pltpu.VMEM((_OUTPAD,), f32),
            pltpu.VMEM((_OUTPAD,), f32),
            pltpu.VMEM((_OUTPAD,), f32),
            pltpu.VMEM((_OUTPAD,), i32),
            pltpu.VMEM((_OUTPAD,), f32),
            pltpu.VMEM_SHARED((1024,), f32),
        ],
    )
    return kern(boxes_t.reshape(-1), scores.reshape(-1), labels.reshape(-1))


def kernel(x, params):
    loc, cls = _ssd_forward(x, params)
    B, N, _ = loc.shape

    boxes_t = jnp.transpose(loc, (2, 0, 1))
    boxes_t = jnp.pad(boxes_t, ((0, 0), (0, 0), (0, _NPAD - N)))
    cls_t = jnp.transpose(cls, (2, 0, 1))
    cls_t = jnp.pad(cls_t, ((0, 0), (0, 0), (0, _NPAD - N)),
                    constant_values=-jnp.inf)

    scores, labels = pl.pallas_call(
        _score_kernel,
        out_shape=(
            jax.ShapeDtypeStruct((B, _NPAD), jnp.float32),
            jax.ShapeDtypeStruct((B, _NPAD), jnp.int32),
        ),
    )(cls_t)

    outs = _run_nms_sc(boxes_t, scores, labels)
    ox1, oy1, ox2, oy2, olab, osc = (o.reshape(_B, _OUTPAD) for o in outs)

    boxes_out = jnp.stack(
        [ox1[:, :_TOPK], oy1[:, :_TOPK], ox2[:, :_TOPK], oy2[:, :_TOPK]],
        axis=-1)
    return boxes_out, olab[:, :_TOPK], osc[:, :_TOPK]
